# baseline (speedup 1.0000x reference)
import functools

import jax
import jax.numpy as jnp
from jax import lax
from jax.experimental import pallas as pl
from jax.experimental.pallas import tpu as pltpu
from jax.experimental.pallas import tpu_sc as plsc

N = 10000
E = 320000
D = 128
R = 8
SEGS = N * R
NC, NS, L = 2, 16, 16
NW = NC * NS
C = 128
ROWS = -(-E // C)
RPW = -(-(-(-ROWS // NW)) // 8) * 8
RPAD = RPW * NW
EPAD = RPAD * C
NPS = N // NS


@functools.lru_cache(maxsize=None)
def _mesh():
    return plsc.VectorSubcoreMesh(
        core_axis_name="c", subcore_axis_name="s", num_cores=NC, num_subcores=NS)


def _wid():
    c = lax.axis_index("c")
    s = lax.axis_index("s")
    return c, s, s * NC + c


def _counts_body(seg_hbm, out_hbm, cnt_v, seg_v):
    _, _, wid = _wid()

    def zero(i, carry):
        cnt_v[pl.ds(i * L, L)] = jnp.zeros((L,), jnp.float32)
        return carry

    lax.fori_loop(0, (SEGS + L) // L, zero, 0)

    pltpu.sync_copy(seg_hbm.at[pl.ds(wid * RPW, RPW)], seg_v)

    ones = jnp.ones((L,), jnp.float32)

    def row(r, carry):
        for i in range(C // L):
            seg16 = seg_v[r, pl.ds(i * L, L)]
            plsc.addupdate_scatter(cnt_v, [seg16], ones)
        return carry

    lax.fori_loop(0, RPW, row, 0)
    pltpu.sync_copy(cnt_v.at[pl.ds(0, SEGS)], out_hbm.at[wid])


@functools.lru_cache(maxsize=None)
def _counts():
    return pl.kernel(
        _counts_body,
        out_type=jax.ShapeDtypeStruct((NW, SEGS), jnp.float32),
        mesh=_mesh(),
        scratch_types=[
            pltpu.VMEM((SEGS + L,), jnp.float32),
            pltpu.VMEM((RPW, C), jnp.int32),
        ],
        compiler_params=pltpu.CompilerParams(needs_layout_passes=False),
    )


def _inv_body(p_ref, o_ref):
    s = jnp.sum(p_ref[...], axis=0)
    o_ref[...] = 0.125 / jnp.maximum(s, 1.0)


def _inv_counts(partial):
    rows = SEGS // D
    out = pl.pallas_call(
        _inv_body,
        out_shape=jax.ShapeDtypeStruct((rows, D), jnp.float32),
    )(partial.reshape(NW, rows, D))
    return out.reshape(SEGS)


HR = RPW // 2
SLOW_CORE = 1


def _ew_body(seg_hbm, ew_hbm, invw_hbm, w_hbm, no_hbm,
             invw_v, seg_v, ew_v, w_v, no_v):
    _, _, wid = _wid()

    pltpu.sync_copy(invw_hbm, invw_v.at[pl.ds(0, SEGS)])
    invw_v[pl.ds(SEGS, L)] = jnp.zeros((L,), jnp.float32)

    nmax = jnp.full((L,), N - 1, jnp.int32)

    for h in range(2):
        base = wid * RPW + h * HR
        pltpu.sync_copy(seg_hbm.at[pl.ds(base, HR)], seg_v)
        pltpu.sync_copy(ew_hbm.at[pl.ds(base, HR)], ew_v)

        def row(r, carry):
            for i in range(C // L):
                sl = pl.ds(i * L, L)
                seg16 = seg_v[r, sl]
                iv = plsc.load_gather(invw_v, [seg16])
                w_v[r, sl] = ew_v[r, sl] * iv
                no_v[r, sl] = jnp.minimum(
                    lax.shift_right_logical(seg16, 3), nmax)
            return carry

        lax.fori_loop(0, HR, row, 0)
        pltpu.sync_copy(w_v, w_hbm.at[pl.ds(base, HR)])
        pltpu.sync_copy(no_v, no_hbm.at[pl.ds(base, HR)])


@functools.lru_cache(maxsize=None)
def _edge_weights():
    return pl.kernel(
        _ew_body,
        out_type=(jax.ShapeDtypeStruct((RPAD, C), jnp.float32),
                  jax.ShapeDtypeStruct((RPAD, C), jnp.int32)),
        mesh=_mesh(),
        scratch_types=[
            pltpu.VMEM((SEGS + L,), jnp.float32),
            pltpu.VMEM((HR, C), jnp.int32),
            pltpu.VMEM((HR, C), jnp.float32),
            pltpu.VMEM((HR, C), jnp.float32),
            pltpu.VMEM((HR, C), jnp.int32),
        ],
        compiler_params=pltpu.CompilerParams(needs_layout_passes=False),
    )


def _agg_body(x_hbm, ni_hbm, w_hbm, no_hbm, out_hbm,
              ni_v, w_v, no_v, rows_v, accum_sh, g0, g1, g2, g3, s0, s1):
    c, s, wid = _wid()
    gsem = ((g0, g1), (g2, g3))
    ssem = (s0, s1)
    CH = C // 2

    def zrow(i, carry):
        for j in range(D // L):
            rows_v[0, i, pl.ds(j * L, L)] = jnp.zeros((L,), jnp.float32)
        return carry

    lax.fori_loop(0, C, zrow, 0)

    @pl.when(s < 10)
    def _():
        for k in range(8):
            sz = 128 if k < 7 else 104
            pltpu.sync_copy(rows_v.at[0].at[pl.ds(0, sz)],
                            accum_sh.at[pl.ds(s * 1000 + k * 128, sz)])

    plsc.subcore_barrier()

    def gather(lc, b):
        for q in range(2):
            pltpu.async_copy(x_hbm.at[ni_v.at[lc, pl.ds(q * CH, CH)]],
                             rows_v.at[b].at[pl.ds(q * CH, CH)], gsem[b][q])

    def gather_wait(lc, b):
        for q in range(2):
            pltpu.make_async_copy(
                x_hbm.at[ni_v.at[lc, pl.ds(q * CH, CH)]],
                rows_v.at[b].at[pl.ds(q * CH, CH)], gsem[b][q]).wait()

    def scatter(lc, b):
        pltpu.async_copy(rows_v.at[b], accum_sh.at[no_v.at[lc]],
                         ssem[b], add=True)

    def scatter_wait(lc, b):
        pltpu.make_async_copy(rows_v.at[b], accum_sh.at[no_v.at[lc]],
                              ssem[b]).wait()

    for h in range(2):

      if True:
        base = wid * RPW + h * HR
        pltpu.sync_copy(ni_hbm.at[pl.ds(base, HR)], ni_v)
        pltpu.sync_copy(w_hbm.at[pl.ds(base, HR)], w_v)
        pltpu.sync_copy(no_hbm.at[pl.ds(base, HR)], no_v)
        gather(0, 0)

        def pair(i, carry):
            for b in range(2):
                lc = 2 * i + b
                gather_wait(lc, b)

                @pl.when(lc >= 1)
                def _():
                    scatter_wait(lc - 1, 1 - b)

                @pl.when(lc < HR - 1)
                def _():
                    gather(lc + 1, 1 - b)

                def sgroup(g, inner):
                    e0 = g * L
                    w16 = w_v[lc, pl.ds(e0, L)]
                    for u in range(L):
                        wb = jnp.take_along_axis(
                            w16, jnp.full((L,), u, jnp.int32), axis=0)
                        for j in range(D // L):
                            sl2 = pl.ds(j * L, L)
                            rows_v[b, e0 + u, sl2] = rows_v[b, e0 + u, sl2] * wb
                    return inner

                lax.fori_loop(0, C // L, sgroup, 0)
                scatter(lc, b)
            return carry

        lax.fori_loop(0, HR // 2, pair, 0)
        scatter_wait(HR - 1, 1)

    plsc.subcore_barrier()

    @pl.when(s < 10)
    def _():
        sl = pl.ds(s * 1000, 1000)
        pltpu.sync_copy(accum_sh.at[sl], out_hbm.at[c].at[sl])


@functools.lru_cache(maxsize=None)
def _aggregate():
    return pl.kernel(
        _agg_body,
        out_type=jax.ShapeDtypeStruct((NC, N, D), jnp.float32),
        mesh=_mesh(),
        scratch_types=[
            pltpu.VMEM((HR, C), jnp.int32),
            pltpu.VMEM((HR, C), jnp.float32),
            pltpu.VMEM((HR, C), jnp.int32),
            pltpu.VMEM((2, C, D), jnp.float32),
            pltpu.VMEM_SHARED((N, D), jnp.float32),
            pltpu.SemaphoreType.DMA,
            pltpu.SemaphoreType.DMA,
            pltpu.SemaphoreType.DMA,
            pltpu.SemaphoreType.DMA,
            pltpu.SemaphoreType.DMA,
            pltpu.SemaphoreType.DMA,
        ],
        compiler_params=pltpu.CompilerParams(needs_layout_passes=False),
    )


def _comb_body(p_ref, o_ref):
    o_ref[...] = p_ref[0] + p_ref[1]


def _combine(parts):
    return pl.pallas_call(
        _comb_body,
        out_shape=jax.ShapeDtypeStruct((N, D), jnp.float32),
        grid=(10,),
        in_specs=[pl.BlockSpec((NC, N // 10, D), lambda i: (0, i, 0))],
        out_specs=pl.BlockSpec((N // 10, D), lambda i: (i, 0)),
    )(parts)


def kernel(input, node_in, node_out, rel, edge_weight):
    x = input
    pad = EPAD - E
    seg = node_out * R + rel
    seg2d = jnp.concatenate(
        [seg, jnp.full((pad,), SEGS, jnp.int32)]).reshape(RPAD, C)
    ni2d = jnp.concatenate(
        [node_in, jnp.zeros((pad,), jnp.int32)]).reshape(RPAD, C)
    ew2d = jnp.concatenate(
        [edge_weight, jnp.zeros((pad,), jnp.float32)]).reshape(RPAD, C)

    partial = _counts()(seg2d)
    invw = _inv_counts(partial)
    w2d, no2d = _edge_weights()(seg2d, ew2d, invw)
    parts = _aggregate()(x, ni2d, w2d, no2d)
    return _combine(parts)


# speedup vs baseline: 1.0008x; 1.0008x over previous
import functools

import jax
import jax.numpy as jnp
from jax import lax
from jax.experimental import pallas as pl
from jax.experimental.pallas import tpu as pltpu
from jax.experimental.pallas import tpu_sc as plsc

N = 10000
E = 320000
D = 128
R = 8
SEGS = N * R
NC, NS, L = 2, 16, 16
NW = NC * NS
C = 128
ROWS = -(-E // C)
RPW = -(-(-(-ROWS // NW)) // 8) * 8
RPAD = RPW * NW
EPAD = RPAD * C
NPS = N // NS


@functools.lru_cache(maxsize=None)
def _mesh():
    return plsc.VectorSubcoreMesh(
        core_axis_name="c", subcore_axis_name="s", num_cores=NC, num_subcores=NS)


def _wid():
    c = lax.axis_index("c")
    s = lax.axis_index("s")
    return c, s, s * NC + c


def _counts_body(seg_hbm, out_hbm, cnt_v, seg_v):
    _, _, wid = _wid()

    def zero(i, carry):
        cnt_v[pl.ds(i * L, L)] = jnp.zeros((L,), jnp.float32)
        return carry

    lax.fori_loop(0, (SEGS + L) // L, zero, 0)

    pltpu.sync_copy(seg_hbm.at[pl.ds(wid * RPW, RPW)], seg_v)

    ones = jnp.ones((L,), jnp.float32)

    def row(r, carry):
        for i in range(C // L):
            seg16 = seg_v[r, pl.ds(i * L, L)]
            plsc.addupdate_scatter(cnt_v, [seg16], ones)
        return carry

    lax.fori_loop(0, RPW, row, 0)
    pltpu.sync_copy(cnt_v.at[pl.ds(0, SEGS)], out_hbm.at[wid])


@functools.lru_cache(maxsize=None)
def _counts():
    return pl.kernel(
        _counts_body,
        out_type=jax.ShapeDtypeStruct((NW, SEGS), jnp.float32),
        mesh=_mesh(),
        scratch_types=[
            pltpu.VMEM((SEGS + L,), jnp.float32),
            pltpu.VMEM((RPW, C), jnp.int32),
        ],
        compiler_params=pltpu.CompilerParams(needs_layout_passes=False),
    )


def _inv_body(p_ref, o_ref):
    s = jnp.sum(p_ref[...], axis=0)
    o_ref[...] = 0.125 / jnp.maximum(s, 1.0)


def _inv_counts(partial):
    rows = SEGS // D
    out = pl.pallas_call(
        _inv_body,
        out_shape=jax.ShapeDtypeStruct((rows, D), jnp.float32),
    )(partial.reshape(NW, rows, D))
    return out.reshape(SEGS)


HR = RPW // 2


def _ew_body(seg_hbm, ew_hbm, invw_hbm, w_hbm, no_hbm,
             invw_v, seg_v, ew_v, w_v, no_v):
    _, _, wid = _wid()

    pltpu.sync_copy(invw_hbm, invw_v.at[pl.ds(0, SEGS)])
    invw_v[pl.ds(SEGS, L)] = jnp.zeros((L,), jnp.float32)

    nmax = jnp.full((L,), N - 1, jnp.int32)

    for h in range(2):
        base = wid * RPW + h * HR
        pltpu.sync_copy(seg_hbm.at[pl.ds(base, HR)], seg_v)
        pltpu.sync_copy(ew_hbm.at[pl.ds(base, HR)], ew_v)

        def row(r, carry):
            for i in range(C // L):
                sl = pl.ds(i * L, L)
                seg16 = seg_v[r, sl]
                iv = plsc.load_gather(invw_v, [seg16])
                w_v[r, sl] = ew_v[r, sl] * iv
                no_v[r, sl] = jnp.minimum(
                    lax.shift_right_logical(seg16, 3), nmax)
            return carry

        lax.fori_loop(0, HR, row, 0)
        pltpu.sync_copy(w_v, w_hbm.at[pl.ds(base, HR)])
        pltpu.sync_copy(no_v, no_hbm.at[pl.ds(base, HR)])


@functools.lru_cache(maxsize=None)
def _edge_weights():
    return pl.kernel(
        _ew_body,
        out_type=(jax.ShapeDtypeStruct((RPAD, C), jnp.float32),
                  jax.ShapeDtypeStruct((RPAD, C), jnp.int32)),
        mesh=_mesh(),
        scratch_types=[
            pltpu.VMEM((SEGS + L,), jnp.float32),
            pltpu.VMEM((HR, C), jnp.int32),
            pltpu.VMEM((HR, C), jnp.float32),
            pltpu.VMEM((HR, C), jnp.float32),
            pltpu.VMEM((HR, C), jnp.int32),
        ],
        compiler_params=pltpu.CompilerParams(needs_layout_passes=False),
    )


def _agg_body(x_hbm, ni_hbm, w_hbm, no_hbm, out_hbm,
              ni_v, w_v, no_v, rows_v, accum_sh, g0, g1, s0, s1):
    c, s, wid = _wid()
    gsem = (g0, g1)
    ssem = (s0, s1)

    def zrow(i, carry):
        for j in range(D // L):
            rows_v[0, i, pl.ds(j * L, L)] = jnp.zeros((L,), jnp.float32)
        return carry

    lax.fori_loop(0, C, zrow, 0)

    @pl.when(s < 10)
    def _():
        for k in range(8):
            sz = 128 if k < 7 else 104
            pltpu.sync_copy(rows_v.at[0].at[pl.ds(0, sz)],
                            accum_sh.at[pl.ds(s * 1000 + k * 128, sz)])

    plsc.subcore_barrier()

    def gather(lc, b):
        pltpu.async_copy(x_hbm.at[ni_v.at[lc]], rows_v.at[b], gsem[b])

    def gather_wait(lc, b):
        pltpu.make_async_copy(
            x_hbm.at[ni_v.at[lc]], rows_v.at[b], gsem[b]).wait()

    def scatter(lc, b):
        pltpu.async_copy(rows_v.at[b], accum_sh.at[no_v.at[lc]],
                         ssem[b], add=True)

    def scatter_wait(lc, b):
        pltpu.make_async_copy(rows_v.at[b], accum_sh.at[no_v.at[lc]],
                              ssem[b]).wait()

    for h in range(2):
        base = wid * RPW + h * HR
        pltpu.sync_copy(ni_hbm.at[pl.ds(base, HR)], ni_v)
        pltpu.sync_copy(w_hbm.at[pl.ds(base, HR)], w_v)
        pltpu.sync_copy(no_hbm.at[pl.ds(base, HR)], no_v)
        gather(0, 0)

        def pair(i, carry):
            for b in range(2):
                lc = 2 * i + b
                gather_wait(lc, b)

                @pl.when(lc >= 1)
                def _():
                    scatter_wait(lc - 1, 1 - b)

                @pl.when(lc < HR - 1)
                def _():
                    gather(lc + 1, 1 - b)

                def sgroup(g, inner):
                    e0 = g * L
                    w16 = w_v[lc, pl.ds(e0, L)]
                    for u in range(L):
                        wb = jnp.take_along_axis(
                            w16, jnp.full((L,), u, jnp.int32), axis=0)
                        for j in range(D // L):
                            sl2 = pl.ds(j * L, L)
                            rows_v[b, e0 + u, sl2] = rows_v[b, e0 + u, sl2] * wb
                    return inner

                lax.fori_loop(0, C // L, sgroup, 0)
                scatter(lc, b)
            return carry

        lax.fori_loop(0, HR // 2, pair, 0)
        scatter_wait(HR - 1, 1)

    plsc.subcore_barrier()

    @pl.when(s < 10)
    def _():
        sl = pl.ds(s * 1000, 1000)
        pltpu.sync_copy(accum_sh.at[sl], out_hbm.at[c].at[sl])


@functools.lru_cache(maxsize=None)
def _aggregate():
    return pl.kernel(
        _agg_body,
        out_type=jax.ShapeDtypeStruct((NC, N, D), jnp.float32),
        mesh=_mesh(),
        scratch_types=[
            pltpu.VMEM((HR, C), jnp.int32),
            pltpu.VMEM((HR, C), jnp.float32),
            pltpu.VMEM((HR, C), jnp.int32),
            pltpu.VMEM((2, C, D), jnp.float32),
            pltpu.VMEM_SHARED((N, D), jnp.float32),
            pltpu.SemaphoreType.DMA,
            pltpu.SemaphoreType.DMA,
            pltpu.SemaphoreType.DMA,
            pltpu.SemaphoreType.DMA,
        ],
        compiler_params=pltpu.CompilerParams(needs_layout_passes=False),
    )


def _comb_body(p_ref, o_ref):
    o_ref[...] = p_ref[0] + p_ref[1]


def _combine(parts):
    return pl.pallas_call(
        _comb_body,
        out_shape=jax.ShapeDtypeStruct((N, D), jnp.float32),
        grid=(10,),
        in_specs=[pl.BlockSpec((NC, N // 10, D), lambda i: (0, i, 0))],
        out_specs=pl.BlockSpec((N // 10, D), lambda i: (i, 0)),
    )(parts)


def kernel(input, node_in, node_out, rel, edge_weight):
    x = input
    pad = EPAD - E
    seg = node_out * R + rel
    seg2d = jnp.concatenate(
        [seg, jnp.full((pad,), SEGS, jnp.int32)]).reshape(RPAD, C)
    ni2d = jnp.concatenate(
        [node_in, jnp.zeros((pad,), jnp.int32)]).reshape(RPAD, C)
    ew2d = jnp.concatenate(
        [edge_weight, jnp.zeros((pad,), jnp.float32)]).reshape(RPAD, C)

    partial = _counts()(seg2d)
    invw = _inv_counts(partial)
    w2d, no2d = _edge_weights()(seg2d, ew2d, invw)
    parts = _aggregate()(x, ni2d, w2d, no2d)
    return _combine(parts)
